# Initial kernel scaffold; baseline (speedup 1.0000x reference)
import jax
import jax.numpy as jnp
from jax.experimental import pallas as pl


def kernel(x, edge_index, W1, b1, W2, b2):
    raise NotImplementedError("write your pallas kernel here")


# speedup vs baseline: 19.6820x; 19.6820x over previous
import functools

import jax
import jax.numpy as jnp
from jax import lax
from jax.experimental import pallas as pl
from jax.experimental.pallas import tpu as pltpu
from jax.experimental.pallas import tpu_sc as plsc

N = 10000
E = 160000
DH = 48
NC = 2
NS = 16
NW = NC * NS
L = 16


_EPT = E // NW
_NFULL = _EPT // L
_REM = _EPT - _NFULL * L

_deg_mesh = plsc.VectorSubcoreMesh(
    core_axis_name="c", subcore_axis_name="s", num_cores=NC, num_subcores=NS)


@functools.partial(
    pl.kernel,
    out_type=jax.ShapeDtypeStruct((NW, N), jnp.float32),
    mesh=_deg_mesh,
    compiler_params=pltpu.CompilerParams(needs_layout_passes=False),
    scratch_types=[
        pltpu.VMEM((N,), jnp.float32),
        pltpu.VMEM((_EPT + 8,), jnp.int32),
    ],
)
def _degree_kernel(dst_hbm, out_hbm, deg_v, idx_v):
    cid = lax.axis_index("c")
    sid = lax.axis_index("s")
    wid = sid * NC + cid

    zeros16 = jnp.zeros((L,), jnp.float32)

    def zero_body(i, _):
        deg_v[pl.ds(i * L, L)] = zeros16
        return 0

    lax.fori_loop(0, N // L, zero_body, 0)

    idx_v[pl.ds(_NFULL * L, L)] = jnp.zeros((L,), jnp.int32)
    pltpu.sync_copy(dst_hbm.at[pl.ds(wid * _EPT, _EPT)],
                    idx_v.at[pl.ds(0, _EPT)])

    ones = jnp.ones((L,), jnp.float32)

    def body(i, _):
        iv = idx_v[pl.ds(i * L, L)]
        plsc.addupdate_scatter(deg_v, [iv], ones)
        return 0

    lax.fori_loop(0, _NFULL, body, 0)
    if _REM:
        iv = idx_v[pl.ds(_NFULL * L, L)]
        lanes = jax.lax.iota(jnp.int32, L)
        plsc.addupdate_scatter(deg_v, [iv], ones, mask=lanes < _REM)

    pltpu.sync_copy(deg_v, out_hbm.at[wid])


_K = 128
_NCHUNK = E // _K
_CPT = -(-_NCHUNK // NW)
_RB = 632
_RB_LAST = N - (NS - 1) * _RB


@functools.partial(
    pl.kernel,
    out_type=jax.ShapeDtypeStruct((NC, N, DH), jnp.float32),
    mesh=plsc.VectorSubcoreMesh(
        core_axis_name="c", subcore_axis_name="s",
        num_cores=NC, num_subcores=NS),
    compiler_params=pltpu.CompilerParams(
        needs_layout_passes=False, use_tc_tiling_on_sc=False),
    scratch_types=[
        pltpu.VMEM_SHARED((N, DH), jnp.float32),
        pltpu.VMEM((_K,), jnp.int32),
        pltpu.VMEM((_K,), jnp.int32),
        pltpu.VMEM((_K, DH), jnp.float32),
        pltpu.SemaphoreType.DMA,
    ],
)
def _scatter_kernel(z_hbm, src_hbm, dst_hbm, zeros_hbm, out_hbm,
                    acc, src_v, dst_v, rows_v, sem):
    cid = lax.axis_index("c")
    sid = lax.axis_index("s")
    wid = sid * NC + cid

    @pl.when(sid < NS - 1)
    def _():
        off = pl.multiple_of(sid * _RB, 8)
        pltpu.sync_copy(zeros_hbm.at[pl.ds(off, _RB)],
                        acc.at[pl.ds(off, _RB)])

    @pl.when(sid == NS - 1)
    def _():
        pltpu.sync_copy(zeros_hbm.at[pl.ds((NS - 1) * _RB, _RB_LAST)],
                        acc.at[pl.ds((NS - 1) * _RB, _RB_LAST)])

    plsc.subcore_barrier()

    def body(j, _):
        gc = j * NW + wid

        @pl.when(gc < _NCHUNK)
        def _():
            base = pl.multiple_of(gc * _K, 8)
            pltpu.sync_copy(src_hbm.at[pl.ds(base, _K)], src_v)
            pltpu.sync_copy(dst_hbm.at[pl.ds(base, _K)], dst_v)
            pltpu.async_copy(z_hbm.at[src_v], rows_v, sem).wait()
            pltpu.sync_copy(rows_v, acc.at[dst_v], add=True)

        return 0

    lax.fori_loop(0, _CPT, body, 0)
    plsc.subcore_barrier()

    @pl.when(sid < NS - 1)
    def _():
        off = pl.multiple_of(sid * _RB, 8)
        pltpu.sync_copy(acc.at[pl.ds(off, _RB)],
                        out_hbm.at[cid, pl.ds(off, _RB)])

    @pl.when(sid == NS - 1)
    def _():
        pltpu.sync_copy(acc.at[pl.ds((NS - 1) * _RB, _RB_LAST)],
                        out_hbm.at[cid, pl.ds((NS - 1) * _RB, _RB_LAST)])


_R = 1000
_G = N // _R


def _tc1_body(x_ref, w1_ref, degp_ref, zp_ref, dis_ref):
    deg = jnp.sum(degp_ref[...], axis=1) + 1.0
    dis = lax.rsqrt(deg)[:, None]
    z = jnp.dot(x_ref[...], w1_ref[...], preferred_element_type=jnp.float32)
    zp_ref[...] = z * dis
    dis_ref[...] = dis


def _tc2_body(sp_ref, zp_ref, dis_ref, b1_ref, u_ref):
    s = sp_ref[0] + sp_ref[1] + zp_ref[...]
    dis = dis_ref[...]
    h = dis * s + b1_ref[...]
    u_ref[...] = dis * jnp.maximum(h, 0.0)


def _tc3_body(tp_ref, u_ref, dis_ref, w2_ref, b2_ref, o_ref):
    v = dis_ref[...] * (tp_ref[0] + tp_ref[1] + u_ref[...])
    o_ref[...] = (jnp.dot(v, w2_ref[...], preferred_element_type=jnp.float32)
                  + b2_ref[...])


def kernel(x, edge_index, W1, b1, W2, b2):
    src = edge_index[0].astype(jnp.int32)
    dst = edge_index[1].astype(jnp.int32)
    zeros2d = jnp.zeros((N, DH), jnp.float32)

    deg_part = jnp.transpose(_degree_kernel(dst))
    f32 = jnp.float32

    zp, dis = pl.pallas_call(
        _tc1_body,
        grid=(_G,),
        in_specs=[
            pl.BlockSpec((_R, 384), lambda i: (i, 0)),
            pl.BlockSpec((384, DH), lambda i: (0, 0)),
            pl.BlockSpec((_R, NW), lambda i: (i, 0)),
        ],
        out_specs=[
            pl.BlockSpec((_R, DH), lambda i: (i, 0)),
            pl.BlockSpec((_R, 1), lambda i: (i, 0)),
        ],
        out_shape=[
            jax.ShapeDtypeStruct((N, DH), f32),
            jax.ShapeDtypeStruct((N, 1), f32),
        ],
    )(x, W1, deg_part)
    del deg_part

    s_part = _scatter_kernel(zp, src, dst, zeros2d)

    u = pl.pallas_call(
        _tc2_body,
        grid=(_G,),
        in_specs=[
            pl.BlockSpec((NC, _R, DH), lambda i: (0, i, 0)),
            pl.BlockSpec((_R, DH), lambda i: (i, 0)),
            pl.BlockSpec((_R, 1), lambda i: (i, 0)),
            pl.BlockSpec((1, DH), lambda i: (0, 0)),
        ],
        out_specs=pl.BlockSpec((_R, DH), lambda i: (i, 0)),
        out_shape=jax.ShapeDtypeStruct((N, DH), f32),
    )(s_part, zp, dis, b1.reshape(1, DH))

    t_part = _scatter_kernel(u, src, dst, zeros2d)

    out = pl.pallas_call(
        _tc3_body,
        grid=(_G,),
        in_specs=[
            pl.BlockSpec((NC, _R, DH), lambda i: (0, i, 0)),
            pl.BlockSpec((_R, DH), lambda i: (i, 0)),
            pl.BlockSpec((_R, 1), lambda i: (i, 0)),
            pl.BlockSpec((DH, 384), lambda i: (0, 0)),
            pl.BlockSpec((1, 384), lambda i: (0, 0)),
        ],
        out_specs=pl.BlockSpec((_R, 384), lambda i: (i, 0)),
        out_shape=jax.ShapeDtypeStruct((N, 384), f32),
    )(t_part, u, dis, W2, b2.reshape(1, 384))

    return out


# speedup vs baseline: 20.3789x; 1.0354x over previous
import functools

import jax
import jax.numpy as jnp
from jax import lax
from jax.experimental import pallas as pl
from jax.experimental.pallas import tpu as pltpu
from jax.experimental.pallas import tpu_sc as plsc

N = 10000
E = 160000
DH = 48
NC = 2
NS = 16
NW = NC * NS
L = 16


_EPT = E // NW
_NFULL = _EPT // L
_REM = _EPT - _NFULL * L

_deg_mesh = plsc.VectorSubcoreMesh(
    core_axis_name="c", subcore_axis_name="s", num_cores=NC, num_subcores=NS)


@functools.partial(
    pl.kernel,
    out_type=jax.ShapeDtypeStruct((NW, N), jnp.float32),
    mesh=_deg_mesh,
    compiler_params=pltpu.CompilerParams(needs_layout_passes=False),
    scratch_types=[
        pltpu.VMEM((N,), jnp.float32),
        pltpu.VMEM((_EPT + 8,), jnp.int32),
    ],
)
def _degree_kernel(dst_hbm, out_hbm, deg_v, idx_v):
    cid = lax.axis_index("c")
    sid = lax.axis_index("s")
    wid = sid * NC + cid

    zeros16 = jnp.zeros((L,), jnp.float32)

    def zero_body(i, _):
        deg_v[pl.ds(i * L, L)] = zeros16
        return 0

    lax.fori_loop(0, N // L, zero_body, 0)

    idx_v[pl.ds(_NFULL * L, L)] = jnp.zeros((L,), jnp.int32)
    pltpu.sync_copy(dst_hbm.at[pl.ds(wid * _EPT, _EPT)],
                    idx_v.at[pl.ds(0, _EPT)])

    ones = jnp.ones((L,), jnp.float32)

    def body(i, _):
        iv = idx_v[pl.ds(i * L, L)]
        plsc.addupdate_scatter(deg_v, [iv], ones)
        return 0

    lax.fori_loop(0, _NFULL, body, 0)
    if _REM:
        iv = idx_v[pl.ds(_NFULL * L, L)]
        lanes = jax.lax.iota(jnp.int32, L)
        plsc.addupdate_scatter(deg_v, [iv], ones, mask=lanes < _REM)

    pltpu.sync_copy(deg_v, out_hbm.at[wid])


_K = 128
_EP = 163840
_NCHUNK = _EP // _K
_CPT = _NCHUNK // NW
_NB = 4
_NPAD = N + 16
_RB = 632
_RB_LAST = N - (NS - 1) * _RB


@functools.partial(
    pl.kernel,
    out_type=jax.ShapeDtypeStruct((NC, N, DH), jnp.float32),
    mesh=plsc.VectorSubcoreMesh(
        core_axis_name="c", subcore_axis_name="s",
        num_cores=NC, num_subcores=NS),
    compiler_params=pltpu.CompilerParams(
        needs_layout_passes=False, use_tc_tiling_on_sc=False),
    scratch_types=[
        pltpu.VMEM_SHARED((_NPAD, DH), jnp.float32),
        [pltpu.VMEM((2, _K), jnp.int32)] * (2 * _NB),
        [pltpu.VMEM((_K, DH), jnp.float32)] * (2 * _NB),
        [pltpu.SemaphoreType.DMA] * (2 * _NB),
        [pltpu.SemaphoreType.DMA] * (2 * _NB),
    ],
)
def _scatter_kernel(z_hbm, ei_hbm, zeros_hbm, out_hbm,
                    acc, idx_v, rows_v, gsem, ssem):
    cid = lax.axis_index("c")
    sid = lax.axis_index("s")
    wid = sid * NC + cid

    @pl.when(sid < NS - 1)
    def _():
        off = pl.multiple_of(sid * _RB, 8)
        pltpu.sync_copy(zeros_hbm.at[pl.ds(off, _RB)],
                        acc.at[pl.ds(off, _RB)])

    @pl.when(sid == NS - 1)
    def _():
        pltpu.sync_copy(zeros_hbm.at[pl.ds((NS - 1) * _RB, _RB_LAST)],
                        acc.at[pl.ds((NS - 1) * _RB, _RB_LAST)])

    plsc.subcore_barrier()

    def start(j, b):
        base = pl.multiple_of((j * NW + wid) * _K, 8)
        pltpu.sync_copy(ei_hbm.at[:, pl.ds(base, _K)], idx_v[b])
        pltpu.async_copy(z_hbm.at[idx_v[b].at[0]], rows_v[b], gsem[b])

    def wait_gather(b):
        pltpu.make_async_copy(z_hbm.at[pl.ds(0, _K)], rows_v[b],
                              gsem[b]).wait()

    def start_scatter(b):
        pltpu.async_copy(rows_v[b], acc.at[idx_v[b].at[1]], ssem[b],
                         add=True)

    def wait_scatter(b):
        pltpu.make_async_copy(rows_v[b], acc.at[pl.ds(0, _K)],
                              ssem[b]).wait()

    for b in range(2 * _NB):
        start(b, b)

    def body(i, _):
        for b in range(2 * _NB):
            wait_gather(b)
            start_scatter(b)
        for b in range(2 * _NB):
            wait_scatter(b)
            start(8 * i + 8 + b, b)
        return 0

    lax.fori_loop(0, _CPT // 8 - 1, body, 0)
    for b in range(2 * _NB):
        wait_gather(b)
        start_scatter(b)
    for b in range(2 * _NB):
        wait_scatter(b)
    plsc.subcore_barrier()

    @pl.when(sid < NS - 1)
    def _():
        off = pl.multiple_of(sid * _RB, 8)
        pltpu.sync_copy(acc.at[pl.ds(off, _RB)],
                        out_hbm.at[cid, pl.ds(off, _RB)])

    @pl.when(sid == NS - 1)
    def _():
        pltpu.sync_copy(acc.at[pl.ds((NS - 1) * _RB, _RB_LAST)],
                        out_hbm.at[cid, pl.ds((NS - 1) * _RB, _RB_LAST)])


_R = 1000
_G = N // _R


def _tc1_body(x_ref, w1_ref, degp_ref, zp_ref, dis_ref):
    deg = jnp.sum(degp_ref[...], axis=1) + 1.0
    dis = lax.rsqrt(deg)[:, None]
    z = jnp.dot(x_ref[...], w1_ref[...], preferred_element_type=jnp.float32)
    zp_ref[...] = z * dis
    dis_ref[...] = dis


def _tc2_body(sp_ref, zp_ref, dis_ref, b1_ref, u_ref):
    s = sp_ref[0] + sp_ref[1] + zp_ref[...]
    dis = dis_ref[...]
    h = dis * s + b1_ref[...]
    u_ref[...] = dis * jnp.maximum(h, 0.0)


def _tc3_body(tp_ref, u_ref, dis_ref, w2_ref, b2_ref, o_ref):
    v = dis_ref[...] * (tp_ref[0] + tp_ref[1] + u_ref[...])
    o_ref[...] = (jnp.dot(v, w2_ref[...], preferred_element_type=jnp.float32)
                  + b2_ref[...])


def kernel(x, edge_index, W1, b1, W2, b2):
    ei = edge_index.astype(jnp.int32)
    dst = ei[1]
    npad = _EP - E
    pad = jnp.stack([jnp.zeros((npad,), jnp.int32),
                     N + (jnp.arange(npad, dtype=jnp.int32) % 16)])
    ei_pad = jnp.concatenate([ei, pad], axis=1)
    zeros2d = jnp.zeros((N, DH), jnp.float32)

    deg_part = jnp.transpose(_degree_kernel(dst))
    f32 = jnp.float32

    zp, dis = pl.pallas_call(
        _tc1_body,
        grid=(_G,),
        in_specs=[
            pl.BlockSpec((_R, 384), lambda i: (i, 0)),
            pl.BlockSpec((384, DH), lambda i: (0, 0)),
            pl.BlockSpec((_R, NW), lambda i: (i, 0)),
        ],
        out_specs=[
            pl.BlockSpec((_R, DH), lambda i: (i, 0)),
            pl.BlockSpec((_R, 1), lambda i: (i, 0)),
        ],
        out_shape=[
            jax.ShapeDtypeStruct((N, DH), f32),
            jax.ShapeDtypeStruct((N, 1), f32),
        ],
    )(x, W1, deg_part)
    del deg_part

    s_part = _scatter_kernel(zp, ei_pad, zeros2d)

    u = pl.pallas_call(
        _tc2_body,
        grid=(_G,),
        in_specs=[
            pl.BlockSpec((NC, _R, DH), lambda i: (0, i, 0)),
            pl.BlockSpec((_R, DH), lambda i: (i, 0)),
            pl.BlockSpec((_R, 1), lambda i: (i, 0)),
            pl.BlockSpec((1, DH), lambda i: (0, 0)),
        ],
        out_specs=pl.BlockSpec((_R, DH), lambda i: (i, 0)),
        out_shape=jax.ShapeDtypeStruct((N, DH), f32),
    )(s_part, zp, dis, b1.reshape(1, DH))

    t_part = _scatter_kernel(u, ei_pad, zeros2d)

    out = pl.pallas_call(
        _tc3_body,
        grid=(_G,),
        in_specs=[
            pl.BlockSpec((NC, _R, DH), lambda i: (0, i, 0)),
            pl.BlockSpec((_R, DH), lambda i: (i, 0)),
            pl.BlockSpec((_R, 1), lambda i: (i, 0)),
            pl.BlockSpec((DH, 384), lambda i: (0, 0)),
            pl.BlockSpec((1, 384), lambda i: (0, 0)),
        ],
        out_specs=pl.BlockSpec((_R, 384), lambda i: (i, 0)),
        out_shape=jax.ShapeDtypeStruct((N, 384), f32),
    )(t_part, u, dis, W2, b2.reshape(1, 384))

    return out


# speedup vs baseline: 31.8175x; 1.5613x over previous
import functools

import jax
import jax.numpy as jnp
from jax import lax
from jax.experimental import pallas as pl
from jax.experimental.pallas import tpu as pltpu
from jax.experimental.pallas import tpu_sc as plsc

N = 10000
E = 160000
DH = 48
NC = 2
NS = 16
NW = NC * NS
L = 16


_EPT = E // NW
_NFULL = _EPT // L
_REM = _EPT - _NFULL * L

_deg_mesh = plsc.VectorSubcoreMesh(
    core_axis_name="c", subcore_axis_name="s", num_cores=NC, num_subcores=NS)


@functools.partial(
    pl.kernel,
    out_type=jax.ShapeDtypeStruct((NW, N), jnp.float32),
    mesh=_deg_mesh,
    compiler_params=pltpu.CompilerParams(needs_layout_passes=False),
    scratch_types=[
        pltpu.VMEM((N,), jnp.float32),
        pltpu.VMEM((_EPT + 8,), jnp.int32),
    ],
)
def _degree_kernel(dst_hbm, out_hbm, deg_v, idx_v):
    cid = lax.axis_index("c")
    sid = lax.axis_index("s")
    wid = sid * NC + cid

    zeros16 = jnp.zeros((L,), jnp.float32)

    def zero_body(i, _):
        deg_v[pl.ds(i * L, L)] = zeros16
        return 0

    lax.fori_loop(0, N // L, zero_body, 0)

    idx_v[pl.ds(_NFULL * L, L)] = jnp.zeros((L,), jnp.int32)
    pltpu.sync_copy(dst_hbm.at[pl.ds(wid * _EPT, _EPT)],
                    idx_v.at[pl.ds(0, _EPT)])

    ones = jnp.ones((L,), jnp.float32)

    def body(i, _):
        iv = idx_v[pl.ds(i * L, L)]
        plsc.addupdate_scatter(deg_v, [iv], ones)
        return 0

    lax.fori_loop(0, _NFULL, body, 0)
    if _REM:
        iv = idx_v[pl.ds(_NFULL * L, L)]
        lanes = jax.lax.iota(jnp.int32, L)
        plsc.addupdate_scatter(deg_v, [iv], ones, mask=lanes < _REM)

    pltpu.sync_copy(deg_v, out_hbm.at[wid])


_K = 128
_EP = 163840
_NCHUNK = _EP // _K
_CPT = _NCHUNK // NW
_NB = 4
_NPAD = N + 16
_RB = 632
_RB_LAST = N - (NS - 1) * _RB


@functools.partial(
    pl.kernel,
    out_type=jax.ShapeDtypeStruct((NC, N, DH), jnp.float32),
    mesh=plsc.VectorSubcoreMesh(
        core_axis_name="c", subcore_axis_name="s",
        num_cores=NC, num_subcores=NS),
    compiler_params=pltpu.CompilerParams(
        needs_layout_passes=False, use_tc_tiling_on_sc=False),
    scratch_types=[
        pltpu.VMEM_SHARED((_NPAD, DH), jnp.float32),
        pltpu.VMEM_SHARED((N, DH), jnp.float32),
        [pltpu.VMEM((2, _K), jnp.int32)] * (2 * _NB),
        [pltpu.VMEM((_K, DH), jnp.float32)] * (2 * _NB),
        [pltpu.SemaphoreType.DMA] * (2 * _NB),
        [pltpu.SemaphoreType.DMA] * (2 * _NB),
    ],
)
def _scatter_kernel(z_hbm, ei_hbm, zeros_hbm, out_hbm,
                    acc, zs, idx_v, rows_v, gsem, ssem):
    cid = lax.axis_index("c")
    sid = lax.axis_index("s")
    wid = sid * NC + cid

    @pl.when(sid < NS - 1)
    def _():
        off = pl.multiple_of(sid * _RB, 8)
        pltpu.sync_copy(zeros_hbm.at[pl.ds(off, _RB)],
                        acc.at[pl.ds(off, _RB)])
        pltpu.sync_copy(z_hbm.at[pl.ds(off, _RB)], zs.at[pl.ds(off, _RB)])

    @pl.when(sid == NS - 1)
    def _():
        off = (NS - 1) * _RB
        pltpu.sync_copy(zeros_hbm.at[pl.ds(off, _RB_LAST)],
                        acc.at[pl.ds(off, _RB_LAST)])
        pltpu.sync_copy(z_hbm.at[pl.ds(off, _RB_LAST)],
                        zs.at[pl.ds(off, _RB_LAST)])

    plsc.subcore_barrier()

    def start(j, b):
        base = pl.multiple_of((j * NW + wid) * _K, 8)
        pltpu.sync_copy(ei_hbm.at[:, pl.ds(base, _K)], idx_v[b])
        pltpu.async_copy(zs.at[idx_v[b].at[0]], rows_v[b], gsem[b])

    def wait_gather(b):
        pltpu.make_async_copy(z_hbm.at[pl.ds(0, _K)], rows_v[b],
                              gsem[b]).wait()

    def start_scatter(b):
        pltpu.async_copy(rows_v[b], acc.at[idx_v[b].at[1]], ssem[b],
                         add=True)

    def wait_scatter(b):
        pltpu.make_async_copy(rows_v[b], acc.at[pl.ds(0, _K)],
                              ssem[b]).wait()

    for b in range(2 * _NB):
        start(b, b)

    def body(i, _):
        for b in range(2 * _NB):
            wait_gather(b)
            start_scatter(b)
        for b in range(2 * _NB):
            wait_scatter(b)
            start(8 * i + 8 + b, b)
        return 0

    lax.fori_loop(0, _CPT // 8 - 1, body, 0)
    for b in range(2 * _NB):
        wait_gather(b)
        start_scatter(b)
    for b in range(2 * _NB):
        wait_scatter(b)
    plsc.subcore_barrier()

    @pl.when(sid < NS - 1)
    def _():
        off = pl.multiple_of(sid * _RB, 8)
        pltpu.sync_copy(acc.at[pl.ds(off, _RB)],
                        out_hbm.at[cid, pl.ds(off, _RB)])

    @pl.when(sid == NS - 1)
    def _():
        pltpu.sync_copy(acc.at[pl.ds((NS - 1) * _RB, _RB_LAST)],
                        out_hbm.at[cid, pl.ds((NS - 1) * _RB, _RB_LAST)])


_R = 1000
_G = N // _R


def _tc1_body(x_ref, w1_ref, degp_ref, zp_ref, dis_ref):
    deg = jnp.sum(degp_ref[...], axis=1) + 1.0
    dis = lax.rsqrt(deg)[:, None]
    z = jnp.dot(x_ref[...], w1_ref[...], preferred_element_type=jnp.float32)
    zp_ref[...] = z * dis
    dis_ref[...] = dis


def _tc2_body(sp_ref, zp_ref, dis_ref, b1_ref, u_ref):
    s = sp_ref[0] + sp_ref[1] + zp_ref[...]
    dis = dis_ref[...]
    h = dis * s + b1_ref[...]
    u_ref[...] = dis * jnp.maximum(h, 0.0)


def _tc3_body(tp_ref, u_ref, dis_ref, w2_ref, b2_ref, o_ref):
    v = dis_ref[...] * (tp_ref[0] + tp_ref[1] + u_ref[...])
    o_ref[...] = (jnp.dot(v, w2_ref[...], preferred_element_type=jnp.float32)
                  + b2_ref[...])


def kernel(x, edge_index, W1, b1, W2, b2):
    ei = edge_index.astype(jnp.int32)
    dst = ei[1]
    npad = _EP - E
    pad = jnp.stack([jnp.zeros((npad,), jnp.int32),
                     N + (jnp.arange(npad, dtype=jnp.int32) % 16)])
    ei_pad = jnp.concatenate([ei, pad], axis=1)
    zeros2d = jnp.zeros((N, DH), jnp.float32)

    deg_part = jnp.transpose(_degree_kernel(dst))
    f32 = jnp.float32

    zp, dis = pl.pallas_call(
        _tc1_body,
        grid=(_G,),
        in_specs=[
            pl.BlockSpec((_R, 384), lambda i: (i, 0)),
            pl.BlockSpec((384, DH), lambda i: (0, 0)),
            pl.BlockSpec((_R, NW), lambda i: (i, 0)),
        ],
        out_specs=[
            pl.BlockSpec((_R, DH), lambda i: (i, 0)),
            pl.BlockSpec((_R, 1), lambda i: (i, 0)),
        ],
        out_shape=[
            jax.ShapeDtypeStruct((N, DH), f32),
            jax.ShapeDtypeStruct((N, 1), f32),
        ],
    )(x, W1, deg_part)
    del deg_part

    s_part = _scatter_kernel(zp, ei_pad, zeros2d)

    u = pl.pallas_call(
        _tc2_body,
        grid=(_G,),
        in_specs=[
            pl.BlockSpec((NC, _R, DH), lambda i: (0, i, 0)),
            pl.BlockSpec((_R, DH), lambda i: (i, 0)),
            pl.BlockSpec((_R, 1), lambda i: (i, 0)),
            pl.BlockSpec((1, DH), lambda i: (0, 0)),
        ],
        out_specs=pl.BlockSpec((_R, DH), lambda i: (i, 0)),
        out_shape=jax.ShapeDtypeStruct((N, DH), f32),
    )(s_part, zp, dis, b1.reshape(1, DH))

    t_part = _scatter_kernel(u, ei_pad, zeros2d)

    out = pl.pallas_call(
        _tc3_body,
        grid=(_G,),
        in_specs=[
            pl.BlockSpec((NC, _R, DH), lambda i: (0, i, 0)),
            pl.BlockSpec((_R, DH), lambda i: (i, 0)),
            pl.BlockSpec((_R, 1), lambda i: (i, 0)),
            pl.BlockSpec((DH, 384), lambda i: (0, 0)),
            pl.BlockSpec((1, 384), lambda i: (0, 0)),
        ],
        out_specs=pl.BlockSpec((_R, 384), lambda i: (i, 0)),
        out_shape=jax.ShapeDtypeStruct((N, 384), f32),
    )(t_part, u, dis, W2, b2.reshape(1, 384))

    return out
